# baseline (speedup 1.0000x reference)
import functools

import jax
import jax.numpy as jnp
from jax import lax
from jax.experimental import pallas as pl
from jax.experimental.pallas import tpu as pltpu
from jax.experimental.pallas import tpu_sc as plsc

_VOCAB = 100000
_N_PREFIX = 20
_D = 64

_NC = 2
_NS = 16
_NW = _NC * _NS

_SEQ = 200
_SUB0 = 128
_SUB1 = _SEQ - _SUB0
_PAD = 208
_SKIP = -1


def _sc_embedding_lookup(embed_weight, new_embed_weight, idx):
    B, S = idx.shape
    assert S == _SEQ and B % (2 * _NW) == 0
    n_b = B // _NW

    mesh = plsc.VectorSubcoreMesh(core_axis_name="c", subcore_axis_name="s")

    @functools.partial(
        pl.kernel,
        mesh=mesh,
        compiler_params=pltpu.CompilerParams(use_tc_tiling_on_sc=False),
        out_type=jax.ShapeDtypeStruct((B, S, _D), jnp.float32),
        scratch_types=[
            pltpu.VMEM((_PAD,), jnp.int32),
            pltpu.VMEM((_PAD,), jnp.int32),
            pltpu.VMEM((2, _SUB0), jnp.int32),
            pltpu.VMEM((2, _SUB0), jnp.int32),
            pltpu.VMEM((2, _SUB0), jnp.int32),
            pltpu.VMEM((2, _SUB0), jnp.int32),
            pltpu.VMEM((_SEQ, _D), jnp.float32),
            pltpu.VMEM((_SEQ, _D), jnp.float32),
            pltpu.SemaphoreType.DMA,
            pltpu.SemaphoreType.DMA,
            pltpu.SemaphoreType.DMA,
            pltpu.SemaphoreType.DMA,
            pltpu.SemaphoreType.DMA,
            pltpu.SemaphoreType.DMA,
        ],
    )
    def k(tab_hbm, pref_hbm, idx_hbm, out_hbm,
          idx_o0, idx_o1, idx_c0, idx_c1, idx_p0, idx_p1, rows0, rows1,
          isem0, isem1, gsem0, gsem1, osem0, osem1):
        wid = lax.axis_index("s") * _NC + lax.axis_index("c")
        b0 = wid * n_b
        bufs = (
            (idx_o0, idx_c0, idx_p0, rows0, isem0, gsem0, osem0),
            (idx_o1, idx_c1, idx_p1, rows1, isem1, gsem1, osem1),
        )

        def idx_cp(i, buf):
            idx_o, _, _, _, isem, _, _ = buf
            return pltpu.make_async_copy(
                idx_hbm.at[b0 + i], idx_o.at[pl.ds(0, _SEQ)], isem
            )

        def out_cp(i, buf):
            rows, osem = buf[3], buf[6]
            return pltpu.make_async_copy(rows, out_hbm.at[b0 + i], osem)

        def main_cps(buf):
            _, idx_c, _, rows, _, gsem, _ = buf
            return (
                pltpu.make_async_copy(
                    tab_hbm.at[idx_c.at[0]],
                    rows.at[pl.ds(0, _SUB0)],
                    gsem,
                ),
                pltpu.make_async_copy(
                    tab_hbm.at[idx_c.at[1, pl.ds(0, _SUB1)]],
                    rows.at[pl.ds(_SUB0, _SUB1)],
                    gsem,
                ),
            )

        def stage_a(i, buf):
            idx_o, idx_c, idx_p, rows, isem, gsem, osem = buf
            idx_cp(i, buf).wait()
            skip = jnp.full((16,), _SKIP, jnp.int32)
            for s in range(_PAD // 16):
                iv = idx_o[pl.ds(s * 16, 16)]
                j, off = (s * 16) // _SUB0, (s * 16) % _SUB0
                idx_c[j, pl.ds(off, 16)] = jnp.minimum(iv, _VOCAB - 1)
                idx_p[j, pl.ds(off, 16)] = jnp.where(
                    iv >= _VOCAB, iv - _VOCAB, skip
                )

            @pl.when(i >= 2)
            def _():
                out_cp(i - 2, buf).wait()

            for cp in main_cps(buf):
                cp.start()

            @pl.when(i + 2 < n_b)
            def _():
                idx_cp(i + 2, buf).start()

        def stage_b(i, buf):
            _, _, idx_p, rows, _, gsem, _ = buf
            for cp in main_cps(buf):
                cp.wait()
            cps = (
                pltpu.async_copy(
                    pref_hbm.at[plsc.Indices(idx_p.at[0], ignored_value=_SKIP)],
                    rows.at[pl.ds(0, _SUB0)],
                    gsem,
                ),
                pltpu.async_copy(
                    pref_hbm.at[
                        plsc.Indices(
                            idx_p.at[1, pl.ds(0, _SUB1)], ignored_value=_SKIP
                        )
                    ],
                    rows.at[pl.ds(_SUB0, _SUB1)],
                    gsem,
                ),
            )
            for cp in cps:
                cp.wait()
            out_cp(i, buf).start()

        idx_cp(0, bufs[0]).start()
        idx_cp(1, bufs[1]).start()

        def pair_body(t, carry):
            i = 2 * t
            stage_a(i, bufs[0])

            @pl.when(t > 0)
            def _():
                stage_b(i - 1, bufs[1])

            stage_a(i + 1, bufs[1])
            stage_b(i, bufs[0])
            return carry

        lax.fori_loop(0, n_b // 2, pair_body, 0)
        stage_b(n_b - 1, bufs[1])
        out_cp(n_b - 2, bufs[0]).wait()
        out_cp(n_b - 1, bufs[1]).wait()

    return k(embed_weight, new_embed_weight, idx)


def kernel(input, embed_weight, new_embed_weight):
    idx = input.astype(jnp.int32)
    return _sc_embedding_lookup(embed_weight, new_embed_weight, idx)


# speedup vs baseline: 1.8136x; 1.8136x over previous
import functools

import jax
import jax.numpy as jnp
from jax import lax
from jax.experimental import pallas as pl
from jax.experimental.pallas import tpu as pltpu
from jax.experimental.pallas import tpu_sc as plsc

_VOCAB = 100000
_N_PREFIX = 20
_D = 64

_NC = 2
_NS = 16
_NW = _NC * _NS

_SUB = 128
_NSUB = 5
_CHUNK = _SUB * _NSUB
_SKIP = -1


def _sc_embedding_lookup(embed_weight, new_embed_weight, idx_flat):
    (B,) = idx_flat.shape
    b_per_w = B // _NW
    n_chunks = b_per_w // _CHUNK
    assert n_chunks % 2 == 0 and n_chunks >= 4

    mesh = plsc.VectorSubcoreMesh(core_axis_name="c", subcore_axis_name="s")

    @functools.partial(
        pl.kernel,
        mesh=mesh,
        compiler_params=pltpu.CompilerParams(use_tc_tiling_on_sc=False),
        out_type=jax.ShapeDtypeStruct((B, 2 * _D), jnp.float32),
        scratch_types=[
            pltpu.VMEM((_CHUNK,), jnp.int32),
            pltpu.VMEM((_CHUNK,), jnp.int32),
            pltpu.VMEM((_NSUB, _SUB), jnp.int32),
            pltpu.VMEM((_NSUB, _SUB), jnp.int32),
            pltpu.VMEM((_NSUB, _SUB), jnp.int32),
            pltpu.VMEM((_NSUB, _SUB), jnp.int32),
            pltpu.VMEM((_CHUNK, _D), jnp.float32),
            pltpu.VMEM((_CHUNK, _D), jnp.float32),
            pltpu.SemaphoreType.DMA,
            pltpu.SemaphoreType.DMA,
            pltpu.SemaphoreType.DMA,
            pltpu.SemaphoreType.DMA,
            pltpu.SemaphoreType.DMA,
            pltpu.SemaphoreType.DMA,
        ],
    )
    def k(tab_hbm, pref_hbm, idx_hbm, out_hbm,
          idx_o0, idx_o1, idx_c0, idx_c1, idx_p0, idx_p1, rows0, rows1,
          isem0, isem1, gsem0, gsem1, osem0, osem1):
        wid = lax.axis_index("s") * _NC + lax.axis_index("c")
        base = wid * b_per_w
        bufs = (
            (idx_o0, idx_c0, idx_p0, rows0, isem0, gsem0, osem0),
            (idx_o1, idx_c1, idx_p1, rows1, isem1, gsem1, osem1),
        )

        def idx_cp(i, buf):
            idx_o, _, _, _, isem, _, _ = buf
            return pltpu.make_async_copy(
                idx_hbm.at[pl.ds(base + i * _CHUNK, _CHUNK)], idx_o, isem
            )

        def out_cp(i, buf):
            rows, osem = buf[3], buf[6]
            return pltpu.make_async_copy(
                rows,
                out_hbm.at[pl.ds(base + i * _CHUNK, _CHUNK), pl.ds(0, _D)],
                osem,
            )

        def main_cp(j, buf):
            _, idx_c, _, rows, _, gsem, _ = buf
            return pltpu.make_async_copy(
                tab_hbm.at[idx_c.at[j]],
                rows.at[pl.ds(j * _SUB, _SUB)],
                gsem,
            )

        def stage_a(i, buf):
            idx_o, idx_c, idx_p, rows, isem, gsem, osem = buf
            idx_cp(i, buf).wait()
            skip = jnp.full((16,), _SKIP, jnp.int32)
            for s in range(_CHUNK // 16):
                iv = idx_o[pl.ds(s * 16, 16)]
                j, off = (s * 16) // _SUB, (s * 16) % _SUB
                idx_c[j, pl.ds(off, 16)] = jnp.minimum(iv, _VOCAB - 1)
                idx_p[j, pl.ds(off, 16)] = jnp.where(
                    iv >= _VOCAB, iv - _VOCAB, skip
                )

            @pl.when(i >= 2)
            def _():
                out_cp(i - 2, buf).wait()

            for j in range(_NSUB):
                main_cp(j, buf).start()

            @pl.when(i + 2 < n_chunks)
            def _():
                idx_cp(i + 2, buf).start()

        def stage_b(i, buf):
            _, _, idx_p, rows, _, gsem, _ = buf
            for j in range(_NSUB):
                main_cp(j, buf).wait()
            cps = [
                pltpu.async_copy(
                    pref_hbm.at[plsc.Indices(idx_p.at[j], ignored_value=_SKIP)],
                    rows.at[pl.ds(j * _SUB, _SUB)],
                    gsem,
                )
                for j in range(_NSUB)
            ]
            for cp in cps:
                cp.wait()
            out_cp(i, buf).start()

        idx_cp(0, bufs[0]).start()
        idx_cp(1, bufs[1]).start()

        def pair_body(t, carry):
            i = 2 * t
            stage_a(i, bufs[0])

            @pl.when(t > 0)
            def _():
                stage_b(i - 1, bufs[1])

            stage_a(i + 1, bufs[1])
            stage_b(i, bufs[0])
            return carry

        lax.fori_loop(0, n_chunks // 2, pair_body, 0)
        stage_b(n_chunks - 1, bufs[1])
        out_cp(n_chunks - 2, bufs[0]).wait()
        out_cp(n_chunks - 1, bufs[1]).wait()

    return k(embed_weight, new_embed_weight, idx_flat)


def kernel(input, embed_weight, new_embed_weight):
    batch, seq = input.shape
    idx_flat = input.reshape(-1).astype(jnp.int32)
    out = _sc_embedding_lookup(embed_weight, new_embed_weight, idx_flat)
    return out.reshape(batch, seq, 2 * _D)[..., :_D]
